# Initial kernel scaffold; baseline (speedup 1.0000x reference)
import jax
import jax.numpy as jnp
from jax.experimental import pallas as pl


def kernel(x, qr, wq_b, wk, ln_w, ln_b, w_weights, position_ids):
    raise NotImplementedError("write your pallas kernel here")


# speedup vs baseline: 1.2470x; 1.2470x over previous
import numpy as np
import jax
import jax.numpy as jnp
from jax import lax
from jax.experimental import pallas as pl
from jax.experimental.pallas import tpu as pltpu
from jax.experimental.pallas import tpu_sc as plsc

S = 2048
DIM = 2048
QR_RANK = 1536
NH = 16
HD = 128
RD = 64
SBLK = 512
NW = 32
ROWS_PER_W = S // NW
NV = S // 16


def _lane_swap(v, h):
    lanes = lax.broadcasted_iota(jnp.int32, v.shape, len(v.shape) - 1)
    bit = lax.bitwise_and(lanes, h)
    fwd = pltpu.roll(v, HD - h, len(v.shape) - 1)
    bwd = pltpu.roll(v, h, len(v.shape) - 1)
    return jnp.where(bit == 0, fwd, bwd)


def _rope_fwht_quant(v, a, b):
    v = v * a + _lane_swap(v, 1) * b
    lanes = lax.broadcasted_iota(jnp.int32, v.shape, len(v.shape) - 1)
    for h in (1, 2, 4, 8, 16, 32, 64):
        sgn = jnp.where(lax.bitwise_and(lanes, h) == 0, 1.0, -1.0)
        v = _lane_swap(v, h) + v * sgn
    v = v * (HD ** -0.5)
    amax = jnp.max(jnp.abs(v), axis=-1, keepdims=True)
    scale = jnp.maximum(amax, 1e-4) / 448.0
    return jnp.clip(v / scale, -448.0, 448.0) * scale


def _transform_body(qr_ref, wqb_ref, x_ref, ww_ref, a_ref, b_ref,
                    qhat_ref, hw_ref):
    a = a_ref[...]
    b = b_ref[...]
    hw_ref[...] = jnp.dot(x_ref[...], ww_ref[...],
                          preferred_element_type=jnp.float32)
    qr = qr_ref[...]
    for h in range(NH):
        qh = jnp.dot(qr, wqb_ref[:, h * HD:(h + 1) * HD],
                     preferred_element_type=jnp.float32)
        qhat_ref[h] = _rope_fwht_quant(qh, a, b)


def _two_sum(a, b):
    s = a + b
    bb = s - a
    err = (a - (s - bb)) + (b - bb)
    return s, err


def _exact4(p0, p1, p2, p3):
    s1, e1 = _two_sum(p0, p1)
    s2, e2 = _two_sum(p2, p3)
    s3, e3 = _two_sum(s1, s2)
    return s3 + ((e1 + e2) + e3)


def _scores_body(qhat_ref, khat_ref, hw_ref, out_ref):
    i = pl.program_id(0)
    j = pl.program_id(1)

    @pl.when(j <= i)
    def _():
        hwb = hw_ref[...].astype(jnp.bfloat16).astype(jnp.float32)
        kb = khat_ref[...].astype(jnp.bfloat16)
        parts = []
        for h in range(NH):
            qb = qhat_ref[h].astype(jnp.bfloat16)
            sh = lax.dot_general(qb, kb, (((1,), (1,)), ((), ())),
                                 preferred_element_type=jnp.float32)
            rh = jnp.maximum(sh.astype(jnp.bfloat16).astype(jnp.float32), 0.0)
            parts.append(rh * hwb[:, h][:, None])
        g = [_exact4(parts[4 * i2], parts[4 * i2 + 1],
                     parts[4 * i2 + 2], parts[4 * i2 + 3]) for i2 in range(4)]
        acc = (g[0] + g[1]) + (g[2] + g[3])
        acc = acc * (HD ** -0.5)
        v = acc + 0.0
        u = lax.bitcast_convert_type(v, jnp.int32)
        m = lax.shift_right_arithmetic(u, 31)
        key_asc = lax.bitwise_xor(u, lax.bitwise_or(m, jnp.int32(-2147483648)))
        out_ref[...] = lax.bitwise_xor(key_asc, jnp.int32(-1))


def _fwht_jnp(x):
    d = x.shape[-1]
    h = 1
    while h < d:
        x = x.reshape(x.shape[:-1] + (d // (2 * h), 2, h))
        a = x[..., 0, :]
        b = x[..., 1, :]
        x = jnp.stack([a + b, a - b], axis=-2)
        x = x.reshape(x.shape[:-3] + (d,))
        h *= 2
    return x


def _rope_jnp(x, cos, sin, rot_end):
    rot = x[..., :rot_end]
    rest = x[..., rot_end:]
    x1 = rot[..., 0::2]
    x2 = rot[..., 1::2]
    o1 = x1 * cos - x2 * sin
    o2 = x1 * sin + x2 * cos
    out = jnp.stack([o1, o2], axis=-1).reshape(rot.shape)
    return jnp.concatenate([out, rest], axis=-1)


def _quant_jnp(x, block=128):
    shp = x.shape
    xb = x.reshape(shp[:-1] + (shp[-1] // block, block))
    amax = jnp.max(jnp.abs(xb), axis=-1, keepdims=True)
    scale = jnp.maximum(amax, 1e-4) / 448.0
    q = jnp.clip(xb / scale, -448.0, 448.0)
    return (q * scale).reshape(shp)


def _dense_keys(x, qr, wq_b, wk, ln_w, ln_b, w_weights, position_ids):
    posf = position_ids.astype(jnp.float32)
    inv_freq = 1.0 / (10000.0 ** (jnp.arange(0, RD, 2, dtype=jnp.float32) / RD))
    ang = posf[:, None] * inv_freq[None, :]
    cos = jnp.cos(ang)
    sin = jnp.sin(ang)

    k = x @ wk

    def _rowsum(v):
        acc = v[:, 0:8]
        for g in range(1, 16):
            acc = acc + v[:, 8 * g:8 * g + 8]
        acc = acc[:, 0:4] + acc[:, 4:8]
        acc = acc[:, 0:2] + acc[:, 2:4]
        return (acc[:, 0] + acc[:, 1])[:, None]

    mu = _rowsum(k) * (1.0 / HD)
    d0 = k - mu
    var = _rowsum(d0 * d0) * (1.0 / HD)
    k = d0 / jnp.sqrt(var + 1e-6) * ln_w + ln_b
    k = _rope_jnp(k, cos, sin, RD)
    k = _fwht_jnp(k) * (HD ** -0.5)
    khat = _quant_jnp(k)

    a_rot = jnp.stack([cos, cos], axis=-1).reshape(S, RD)
    b_rot = jnp.stack([-sin, sin], axis=-1).reshape(S, RD)
    a_tab = jnp.concatenate([a_rot, jnp.ones((S, HD - RD), jnp.float32)], axis=1)
    b_tab = jnp.concatenate([b_rot, jnp.zeros((S, HD - RD), jnp.float32)], axis=1)

    nblk = S // SBLK
    qhat, hw = pl.pallas_call(
        _transform_body,
        grid=(nblk,),
        in_specs=[
            pl.BlockSpec((SBLK, QR_RANK), lambda i: (i, 0)),
            pl.BlockSpec((QR_RANK, NH * HD), lambda i: (0, 0)),
            pl.BlockSpec((SBLK, DIM), lambda i: (i, 0)),
            pl.BlockSpec((DIM, NH), lambda i: (0, 0)),
            pl.BlockSpec((SBLK, HD), lambda i: (i, 0)),
            pl.BlockSpec((SBLK, HD), lambda i: (i, 0)),
        ],
        out_specs=[
            pl.BlockSpec((NH, SBLK, HD), lambda i: (0, i, 0)),
            pl.BlockSpec((SBLK, NH), lambda i: (i, 0)),
        ],
        out_shape=[
            jax.ShapeDtypeStruct((NH, S, HD), jnp.float32),
            jax.ShapeDtypeStruct((S, NH), jnp.float32),
        ],
    )(qr, wq_b, x, w_weights, a_tab, b_tab)

    keys = pl.pallas_call(
        _scores_body,
        grid=(nblk, nblk),
        in_specs=[
            pl.BlockSpec((NH, SBLK, HD), lambda i, j: (0, i, 0)),
            pl.BlockSpec((SBLK, HD), lambda i, j: (j, 0)),
            pl.BlockSpec((SBLK, NH), lambda i, j: (i, 0)),
        ],
        out_specs=pl.BlockSpec((SBLK, SBLK), lambda i, j: (i, j)),
        out_shape=jax.ShapeDtypeStruct((S, S), jnp.int32),
    )(qhat, khat, hw)
    return keys


def _radix_pass(shift, n, nv, src_k, src_i, dst_k, dst_i, hist):
    def clr(i, c):
        hist[pl.ds(i * 16, 16)] = jnp.zeros((16,), jnp.int32)
        return c

    lax.fori_loop(0, 16, clr, 0, unroll=True)

    def histb(i, c):
        kx = src_k[pl.ds(i * 16, 16)]
        d = lax.bitwise_and(lax.shift_right_logical(kx, shift), 255)
        cnt, last = plsc.scan_count(d)
        plsc.addupdate_scatter(hist, [d], cnt, mask=last)
        return c

    lax.fori_loop(0, nv, histb, 0)

    def scan(i, run):
        v = hist[pl.ds(i * 16, 16)]
        inc = plsc.cumsum(v)
        hist[pl.ds(i * 16, 16)] = inc - v + run
        return run + jnp.sum(v)

    lax.fori_loop(0, 16, scan, jnp.int32(0), unroll=True)

    def perm(i, c):
        kx = src_k[pl.ds(i * 16, 16)]
        ix = src_i[pl.ds(i * 16, 16)]
        d = lax.bitwise_and(lax.shift_right_logical(kx, shift), 255)
        cnt, last = plsc.scan_count(d)
        base = plsc.load_gather(hist, [d])
        pos = base + cnt - 1
        plsc.store_scatter(dst_k, [pos], kx)
        plsc.store_scatter(dst_i, [pos], ix)
        plsc.addupdate_scatter(hist, [d], cnt, mask=last)
        return c

    lax.fori_loop(0, nv, perm, 0)


def _sc_sort_body(keys_hbm, out_hbm, kb0, kb1, ib0, ib1, hist):
    core = lax.axis_index("c")
    sub = lax.axis_index("s")
    w = sub * 2 + core
    iota = lax.iota(jnp.int32, 16)

    def row_body(r, c):
        s = w + r * NW
        n = s + 1
        nv = (n + 15) // 16
        pltpu.sync_copy(keys_hbm.at[s], kb0)
        base = (nv - 1) * 16
        kv = kb0[pl.ds(base, 16)]
        kb0[pl.ds(base, 16)] = jnp.where(iota + base < n, kv, jnp.int32(-1))

        def initi(i, c2):
            ib0[pl.ds(i * 16, 16)] = iota + i * 16
            return c2

        lax.fori_loop(0, nv, initi, 0)

        _radix_pass(0, n, nv, kb0, ib0, kb1, ib1, hist)
        _radix_pass(8, n, nv, kb1, ib1, kb0, ib0, hist)
        _radix_pass(16, n, nv, kb0, ib0, kb1, ib1, hist)
        _radix_pass(24, n, nv, kb1, ib1, kb0, ib0, hist)

        def tail(i, c2):
            posv = iota + i * 16
            plsc.store_scatter(ib0, [posv], posv, mask=posv >= n)
            return c2

        lax.fori_loop(nv - 1, NV, tail, 0)
        pltpu.sync_copy(ib0, out_hbm.at[s])
        return c

    lax.fori_loop(0, ROWS_PER_W, row_body, 0)


def _sc_sort(keys):
    mesh = plsc.VectorSubcoreMesh(core_axis_name="c", subcore_axis_name="s")
    f = pl.kernel(
        _sc_sort_body,
        out_type=jax.ShapeDtypeStruct((S, S), jnp.int32),
        mesh=mesh,
        scratch_types=[
            pltpu.VMEM((S,), jnp.int32),
            pltpu.VMEM((S,), jnp.int32),
            pltpu.VMEM((S,), jnp.int32),
            pltpu.VMEM((S,), jnp.int32),
            pltpu.VMEM((256,), jnp.int32),
        ],
        compiler_params=pltpu.CompilerParams(needs_layout_passes=False),
    )
    return f(keys)


def kernel(x, qr, wq_b, wk, ln_w, ln_b, w_weights, position_ids):
    keys = _dense_keys(x, qr, wq_b, wk, ln_w, ln_b, w_weights, position_ids)
    return _sc_sort(keys)
